# baseline (speedup 1.0000x reference)
import jax
import jax.numpy as jnp
from jax import lax
from jax.experimental import pallas as pl
from jax.experimental.pallas import tpu as pltpu
from jax.experimental.pallas import tpu_sc as plsc

D_MODEL = 64
PAD_W = 128
NUM_WORKERS = 32
LANE = 128
NBUF = 4
B_DIM = 4096
T_DIM = 200
VOCAB_N = 1000000
TBLK = 16384


def _transpose_kernel(src_ref, dst_ref):
    t = src_ref[...].T
    dst_ref[...] = jnp.concatenate([t, jnp.zeros_like(t)], axis=1)


def _gather_kernel(dense_hbm, xt_hbm, out_hbm, idx_v,
                   r0, r1, r2, r3, t0, t1,
                   g0, g1, g2, g3, s0, s1):
    wid = lax.axis_index("s") * 2 + lax.axis_index("c")
    rows = [r0, r1, r2, r3]
    tb = [t0, t1]
    gsem = [g0, g1, g2, g3]
    ssem = [s0, s1]

    pltpu.sync_copy(xt_hbm.at[:, pl.ds(wid * LANE, LANE)], idx_v)

    def fire_gather(chunk, b):
        pltpu.async_copy(dense_hbm.at[idx_v.at[chunk]], rows[b], gsem[b])

    def wait_gather(b):
        pltpu.make_async_copy(dense_hbm.at[pl.ds(0, LANE), :], rows[b], gsem[b]).wait()

    def transpose_chunk(b, e):
        def tbody(d, carry):
            col = jnp.full((16,), 0, jnp.int32) + d
            for g in range(8):
                toks = lax.iota(jnp.int32, 16) + 16 * g
                vals = plsc.load_gather(rows[b], [toks, col])
                tb[e][d // 8, d % 8, pl.ds(16 * g, 16)] = vals
            return carry

        lax.fori_loop(0, D_MODEL, tbody, 0, unroll=False)

    def fire_stores(chunk, e):
        for i in range(8):
            pltpu.async_copy(tb[e].at[i], out_hbm.at[chunk, i, wid], ssem[e])

    def wait_stores(chunk, e):
        pltpu.make_async_copy(tb[e], out_hbm.at[chunk, :, wid], ssem[e]).wait()

    def step(chunk, i, wait_tb, refill):
        e = i % 2
        wait_gather(i)
        if wait_tb:
            wait_stores(chunk, e)
        transpose_chunk(i, e)
        fire_stores(chunk, e)
        if refill:
            fire_gather(chunk + 3, (i + 3) % NBUF)

    for b in range(3):
        fire_gather(b, b)

    step(0, 0, False, True)
    step(1, 1, False, True)
    step(2, 2, True, True)
    step(3, 3, True, True)

    def body(t, carry):
        for i in range(NBUF):
            step(NBUF * t + i, i, True, True)
        return carry

    lax.fori_loop(1, T_DIM // NBUF - 1, body, 0, unroll=False)

    tail = T_DIM - NBUF
    step(tail, 0, True, True)
    step(tail + 1, 1, True, False)
    step(tail + 2, 2, True, False)
    step(tail + 3, 3, True, False)

    wait_stores(0, 0)
    wait_stores(1, 1)


@jax.jit
def _embed(xt, table_t):
    nblk = (VOCAB_N + TBLK - 1) // TBLK
    dense = pl.pallas_call(
        _transpose_kernel,
        grid=(nblk,),
        in_specs=[pl.BlockSpec((D_MODEL, TBLK), lambda i: (0, i))],
        out_specs=pl.BlockSpec((TBLK, PAD_W), lambda i: (i, 0)),
        out_shape=jax.ShapeDtypeStruct((VOCAB_N, PAD_W), jnp.float32),
    )(table_t)

    mesh = plsc.VectorSubcoreMesh(core_axis_name="c", subcore_axis_name="s")
    gather_run = pl.kernel(
        _gather_kernel,
        out_type=jax.ShapeDtypeStruct((T_DIM, 8, NUM_WORKERS, 8, LANE), jnp.float32),
        mesh=mesh,
        scratch_types=[
            pltpu.VMEM((T_DIM, LANE), jnp.int32),
            pltpu.VMEM((LANE, PAD_W), jnp.float32),
            pltpu.VMEM((LANE, PAD_W), jnp.float32),
            pltpu.VMEM((LANE, PAD_W), jnp.float32),
            pltpu.VMEM((LANE, PAD_W), jnp.float32),
            pltpu.VMEM((8, 8, LANE), jnp.float32),
            pltpu.VMEM((8, 8, LANE), jnp.float32),
            pltpu.SemaphoreType.DMA,
            pltpu.SemaphoreType.DMA,
            pltpu.SemaphoreType.DMA,
            pltpu.SemaphoreType.DMA,
            pltpu.SemaphoreType.DMA,
            pltpu.SemaphoreType.DMA,
        ],
        compiler_params=pltpu.CompilerParams(use_tc_tiling_on_sc=True, needs_layout_passes=False),
    )
    return gather_run(dense, xt)


def kernel(x, table):
    phys = _embed(x.T.astype(jnp.int32), table.T)
    return phys.transpose(2, 4, 0, 1, 3).reshape(B_DIM, T_DIM, D_MODEL)


# speedup vs baseline: 2.0356x; 2.0356x over previous
import jax
import jax.numpy as jnp
from jax import lax
from jax.experimental import pallas as pl
from jax.experimental.pallas import tpu as pltpu
from jax.experimental.pallas import tpu_sc as plsc

D_MODEL = 64
PAD_W = 128
NUM_WORKERS = 32
LANE = 128
NBUF = 4
TOTAL = 4096 * 200
PER_WORKER = TOTAL // NUM_WORKERS
IDX_ROWS = PER_WORKER // LANE
VOCAB_N = 1000000
TBLK = 32768


def _transpose_kernel(src_ref, dst_ref):
    t = src_ref[...].T
    dst_ref[...] = jnp.concatenate([t, jnp.zeros_like(t)], axis=1)


def _gather_kernel(dense_hbm, idx_hbm, out_hbm, idx_v,
                   r0, r1, r2, r3, g0, g1, g2, g3, s0, s1, s2, s3):
    wid = lax.axis_index("s") * 2 + lax.axis_index("c")
    base = wid * PER_WORKER
    rows = [r0, r1, r2, r3]
    gsem = [g0, g1, g2, g3]
    ssem = [s0, s1, s2, s3]

    pltpu.sync_copy(idx_hbm.at[wid], idx_v)

    def fire_gather(chunk, b):
        pltpu.async_copy(dense_hbm.at[idx_v.at[chunk]], rows[b], gsem[b])

    def wait_gather(chunk, b):
        pltpu.make_async_copy(
            out_hbm.at[pl.ds(base + chunk * LANE, LANE), :], rows[b], gsem[b]
        ).wait()

    def fire_store(chunk, b):
        pltpu.async_copy(
            rows[b], out_hbm.at[pl.ds(base + chunk * LANE, LANE), :], ssem[b]
        )

    def wait_store(chunk, b):
        pltpu.make_async_copy(
            rows[b], out_hbm.at[pl.ds(base + chunk * LANE, LANE), :], ssem[b]
        ).wait()

    def step(chunk, i, refill, fresh):
        wait_gather(chunk, i)
        fire_store(chunk, i)
        bn = (i + 3) % NBUF
        if refill:
            if not fresh:
                wait_store(chunk, bn)
            fire_gather(chunk + 3, bn)

    for b in range(3):
        fire_gather(b, b)

    step(0, 0, True, True)
    step(1, 1, True, False)
    step(2, 2, True, False)
    step(3, 3, True, False)

    def body(t, carry):
        for i in range(NBUF):
            chunk = NBUF * t + i
            wait_gather(chunk, i)
            fire_store(chunk, i)
            bn = (i + 3) % NBUF
            wait_store(chunk, bn)
            fire_gather(chunk + 3, bn)
        return carry

    lax.fori_loop(1, IDX_ROWS // NBUF - 1, body, 0, unroll=False)

    tail = IDX_ROWS - NBUF
    step(tail, 0, True, False)
    step(tail + 1, 1, False, False)
    step(tail + 2, 2, False, False)
    step(tail + 3, 3, False, False)

    for b in range(NBUF):
        wait_store(b, b)


@jax.jit
def _embed(x_blocked, table_t):
    nblk = (VOCAB_N + TBLK - 1) // TBLK
    dense = pl.pallas_call(
        _transpose_kernel,
        grid=(nblk,),
        in_specs=[pl.BlockSpec((D_MODEL, TBLK), lambda i: (0, i))],
        out_specs=pl.BlockSpec((TBLK, PAD_W), lambda i: (i, 0)),
        out_shape=jax.ShapeDtypeStruct((VOCAB_N, PAD_W), jnp.float32),
    )(table_t)

    mesh = plsc.VectorSubcoreMesh(core_axis_name="c", subcore_axis_name="s")
    gather_run = pl.kernel(
        _gather_kernel,
        out_type=jax.ShapeDtypeStruct((TOTAL, PAD_W), jnp.float32),
        mesh=mesh,
        scratch_types=[
            pltpu.VMEM((IDX_ROWS, LANE), jnp.int32),
            pltpu.VMEM((LANE, PAD_W), jnp.float32),
            pltpu.VMEM((LANE, PAD_W), jnp.float32),
            pltpu.VMEM((LANE, PAD_W), jnp.float32),
            pltpu.VMEM((LANE, PAD_W), jnp.float32),
            pltpu.SemaphoreType.DMA,
            pltpu.SemaphoreType.DMA,
            pltpu.SemaphoreType.DMA,
            pltpu.SemaphoreType.DMA,
            pltpu.SemaphoreType.DMA,
            pltpu.SemaphoreType.DMA,
            pltpu.SemaphoreType.DMA,
            pltpu.SemaphoreType.DMA,
        ],
        compiler_params=pltpu.CompilerParams(use_tc_tiling_on_sc=True, needs_layout_passes=False),
    )
    return gather_run(dense, x_blocked)


def kernel(x, table):
    b, t = x.shape
    x_blocked = x.reshape(NUM_WORKERS, IDX_ROWS, LANE).astype(jnp.int32)
    out_padded = _embed(x_blocked, table.T)
    return out_padded[:, :D_MODEL].reshape(b, t, D_MODEL)
